# baseline (speedup 1.0000x reference)
import jax
import jax.numpy as jnp
import numpy as np
from jax.experimental import pallas as pl
from jax.experimental.pallas import tpu as pltpu

ROWS = 128
COLS = 100000
BR = 8
BC = 16384
CH = 128
NCH = BC // CH
NCB = (COLS + BC - 1) // BC

_U32 = jnp.uint32
_NEG_INF = np.float32(-np.inf)
_NEG_LN2 = np.float32(-0.6931471805599453)


def _threefry_gumbel(x1):
    k0 = np.uint32(0)
    k1 = np.uint32(42)
    ks = (k0, k1, np.uint32(k0 ^ k1 ^ np.uint32(0x1BD11BDA)))
    rot = ((13, 15, 26, 6), (17, 29, 16, 24))

    x0 = x1
    x1 = (x1 << _U32(13)) | (x1 >> _U32(19))
    x1 = x1 ^ x0
    first = True
    for n in range(5):
        for r in rot[n % 2]:
            if first:
                first = False
                continue
            x0 = x0 + x1
            x1 = (x1 << _U32(r)) | (x1 >> _U32(32 - r))
            x1 = x1 ^ x0
        x0 = x0 + ks[(n + 1) % 3]
        x1 = x1 + ks[(n + 2) % 3] + _U32(n + 1)
    bits = x0 ^ x1

    fl = jax.lax.bitcast_convert_type(
        (bits >> _U32(9)) | _U32(0x3F800000), jnp.float32
    )
    u = fl - np.float32(1.0)
    t = jnp.log2(u) * _NEG_LN2
    return jnp.log2(t) * _NEG_LN2


def _body(x_ref, out_ref, bestv, besti):
    r = pl.program_id(0)
    c = pl.program_id(1)

    lane = jax.lax.broadcasted_iota(jnp.int32, (BR, CH), 1)
    row = r * BR + jax.lax.broadcasted_iota(jnp.int32, (BR, CH), 0)
    base42 = row * COLS + lane + 42
    limit = (row + 1) * COLS + 42
    cbase = c * BC

    bv = jnp.full((BR, CH), _NEG_INF, jnp.float32)
    bi = jnp.zeros((BR, CH), jnp.int32)
    for j in range(NCH):
        cnt = base42 + (cbase + j * CH)
        g = _threefry_gumbel(cnt.astype(_U32))
        val = x_ref[:, j * CH:(j + 1) * CH] + g
        val = jnp.where(cnt < limit, val, _NEG_INF)
        upd = val > bv
        bv = jnp.where(upd, val, bv)
        bi = jnp.where(upd, cnt, bi)

    @pl.when(c == 0)
    def _init():
        bestv[...] = bv
        besti[...] = bi

    @pl.when(c != 0)
    def _fold():
        ov = bestv[...]
        oi = besti[...]
        upd = bv > ov
        nv = jnp.where(upd, bv, ov)
        bestv[...] = nv
        besti[...] = jnp.where(upd, bi, oi)

        @pl.when(c == NCB - 1)
        def _emit():
            ni = jnp.where(upd, bi, oi)
            m = jnp.max(nv, axis=1, keepdims=True)
            cand = jnp.where(nv == m, ni, jnp.int32(0x7FFFFFFF))
            win = jnp.min(cand, axis=1, keepdims=True)
            rowv = r * BR + jax.lax.broadcasted_iota(jnp.int32, (BR, 1), 0)
            out_ref[...] = win - (rowv * COLS + 42)


@jax.jit
def _run(outputs):
    out = pl.pallas_call(
        _body,
        grid=(ROWS // BR, NCB),
        in_specs=[pl.BlockSpec((BR, BC), lambda r, c: (r, c))],
        out_specs=pl.BlockSpec((BR, 1), lambda r, c: (r, 0)),
        out_shape=jax.ShapeDtypeStruct((ROWS, 1), jnp.int32),
        scratch_shapes=[
            pltpu.VMEM((BR, CH), jnp.float32),
            pltpu.VMEM((BR, CH), jnp.int32),
        ],
        compiler_params=pltpu.CompilerParams(
            dimension_semantics=("parallel", "arbitrary"),
        ),
    )(outputs)
    return out[:, 0]


def kernel(outputs):
    return _run(outputs)


# speedup vs baseline: 1.0667x; 1.0667x over previous
import jax
import jax.numpy as jnp
import numpy as np
from jax.experimental import pallas as pl
from jax.experimental.pallas import tpu as pltpu

ROWS = 128
COLS = 100000
BR = 8
BC = 8192
CH = 128
NCH = BC // CH
NCB = (COLS + BC - 1) // BC

_U32 = jnp.uint32
_NEG_INF = np.float32(-np.inf)


def _threefry_gumbel(x1):
    k0 = np.uint32(0)
    k1 = np.uint32(42)
    ks = (k0, k1, np.uint32(k0 ^ k1 ^ np.uint32(0x1BD11BDA)))
    rot = ((13, 15, 26, 6), (17, 29, 16, 24))

    x0 = x1
    x1 = (x1 << _U32(13)) | (x1 >> _U32(19))
    x1 = x1 ^ x0
    first = True
    for n in range(5):
        for r in rot[n % 2]:
            if first:
                first = False
                continue
            x0 = x0 + x1
            x1 = (x1 << _U32(r)) | (x1 >> _U32(32 - r))
            x1 = x1 ^ x0
        x0 = x0 + ks[(n + 1) % 3]
        x1 = x1 + ks[(n + 2) % 3] + _U32(n + 1)
    bits = x0 ^ x1

    fl = jax.lax.bitcast_convert_type(
        (bits >> _U32(9)) | _U32(0x3F800000), jnp.float32
    )
    u = fl - np.float32(1.0)
    return jnp.log(-jnp.log(u))


def _body(x_ref, out_ref, bestv, besti):
    r = pl.program_id(0)
    c = pl.program_id(1)

    lane = jax.lax.broadcasted_iota(jnp.int32, (BR, CH), 1)
    row = r * BR + jax.lax.broadcasted_iota(jnp.int32, (BR, CH), 0)
    base42 = row * COLS + lane + 42
    limit = (row + 1) * COLS + 42
    cbase = c * BC

    bv = jnp.full((BR, CH), _NEG_INF, jnp.float32)
    bi = jnp.zeros((BR, CH), jnp.int32)
    for j in range(NCH):
        cnt = base42 + (cbase + j * CH)
        nlg = _threefry_gumbel(cnt.astype(_U32))
        val = x_ref[:, j * CH:(j + 1) * CH] - nlg
        upd = (val > bv) & (cnt < limit)
        bv = jnp.where(upd, val, bv)
        bi = jnp.where(upd, cnt, bi)

    @pl.when(c == 0)
    def _init():
        bestv[...] = bv
        besti[...] = bi

    @pl.when(c != 0)
    def _fold():
        ov = bestv[...]
        oi = besti[...]
        upd = bv > ov
        nv = jnp.where(upd, bv, ov)
        bestv[...] = nv
        besti[...] = jnp.where(upd, bi, oi)

        @pl.when(c == NCB - 1)
        def _emit():
            ni = jnp.where(upd, bi, oi)
            m = jnp.max(nv, axis=1, keepdims=True)
            cand = jnp.where(nv == m, ni, jnp.int32(0x7FFFFFFF))
            win = jnp.min(cand, axis=1, keepdims=True)
            rowv = r * BR + jax.lax.broadcasted_iota(jnp.int32, (BR, 1), 0)
            out_ref[...] = win - (rowv * COLS + 42)


@jax.jit
def _run(outputs):
    out = pl.pallas_call(
        _body,
        grid=(ROWS // BR, NCB),
        in_specs=[pl.BlockSpec((BR, BC), lambda r, c: (r, c))],
        out_specs=pl.BlockSpec((BR, 1), lambda r, c: (r, 0)),
        out_shape=jax.ShapeDtypeStruct((ROWS, 1), jnp.int32),
        scratch_shapes=[
            pltpu.VMEM((BR, CH), jnp.float32),
            pltpu.VMEM((BR, CH), jnp.int32),
        ],
        compiler_params=pltpu.CompilerParams(
            dimension_semantics=("parallel", "arbitrary"),
        ),
    )(outputs)
    return out[:, 0]


def kernel(outputs):
    return _run(outputs)
